# baseline (speedup 1.0000x reference)
import functools

import jax
import jax.numpy as jnp
from jax import lax
from jax.experimental import pallas as pl
from jax.experimental.pallas import tpu as pltpu
from jax.experimental.pallas import tpu_sc as plsc

N = 50000
E = 800000
H = 64

HALF = 25088
NPAD = 2 * HALF
RPT = HALF // 16
TRASH = HALF
ACCR = HALF + 8
C = 128
PT = E // 16
NCH = (PT + C - 1) // C
EPAD = E + 2 * C
FILLDST = 1 << 30

f32 = jnp.float32
i32 = jnp.int32


@functools.lru_cache(maxsize=None)
def _make_edge_pass(with_deg):
  mesh = plsc.VectorSubcoreMesh(
      core_axis_name="c", subcore_axis_name="s", num_cores=2, num_subcores=16)

  out_type = [jax.ShapeDtypeStruct((NPAD, H), f32)]
  scratch = [
      pltpu.VMEM((H,), f32),
      pltpu.VMEM((H,), f32),
      pltpu.VMEM((C,), i32),
      pltpu.VMEM((C,), i32),
      pltpu.VMEM((C,), i32),
      pltpu.VMEM((C,), i32),
      pltpu.VMEM((C, H), f32),
      pltpu.VMEM((C, H), f32),
      pltpu.VMEM_SHARED((ACCR, H), f32),
      pltpu.SemaphoreType.DMA,
      pltpu.SemaphoreType.DMA,
  ]
  if with_deg:
    out_type.append(jax.ShapeDtypeStruct((NPAD, H), f32))

  def body(a_h, b_h, sdst_h, ssrc_h, sc_h, sh_h, *refs):
    if with_deg:
      (acc_out, deg_out, scv, shv, dstv, srcv, gidxv, sidxv,
       arow, brow, acc, sem_a, sem_b) = refs
    else:
      (acc_out, scv, shv, dstv, srcv, gidxv, sidxv,
       arow, brow, acc, sem_a, sem_b) = refs
      deg_out = None

    c = lax.axis_index("c")
    t = lax.axis_index("s")

    pltpu.sync_copy(sc_h, scv)
    pltpu.sync_copy(sh_h, shv)
    scale = [scv[pl.ds(f * 16, 16)] for f in range(4)]
    shift = [shv[pl.ds(f * 16, 16)] for f in range(4)]

    def fill_arow(val):
      def frow(r, carry):
        for f in range(4):
          arow[r, pl.ds(f * 16, 16)] = jnp.full((16,), val, f32)
        return carry
      lax.fori_loop(0, C, frow, 0)

    zbase = t * RPT

    def zero_acc():
      for i in range(12):
        pltpu.sync_copy(arow, acc.at[pl.ds(zbase + i * C, C)])
      pltpu.sync_copy(arow.at[pl.ds(0, 32)],
                      acc.at[pl.ds(zbase + 12 * C, 32)])

    base = t * PT
    limit = (t + 1) * PT
    chalf = c * HALF
    iot = lax.iota(i32, 16)

    if with_deg:
      fill_arow(0.0)
      zero_acc()
      plsc.subcore_barrier()
      fill_arow(1.0)

      def dchunk(k, carry):
        p0 = pl.multiple_of(base + k * C, 8)
        pltpu.async_copy(sdst_h.at[pl.ds(p0, C)], dstv, sem_a).wait()
        for j in range(C // 16):
          d = dstv[pl.ds(j * 16, 16)]
          loc = d - chalf
          pos = (p0 + j * 16) + iot
          valid = (loc >= 0) & (loc < HALF) & (pos < limit)
          sidxv[pl.ds(j * 16, 16)] = jnp.where(valid, loc, TRASH)
        pltpu.sync_copy(arow, acc.at[sidxv], add=True)
        return carry
      lax.fori_loop(0, NCH, dchunk, 0)

      plsc.subcore_barrier()
      pltpu.sync_copy(acc.at[pl.ds(t * RPT, RPT)],
                      deg_out.at[pl.ds(chalf + t * RPT, RPT)])

    fill_arow(0.0)
    zero_acc()
    plsc.subcore_barrier()

    def chunk(k, carry):
      p0 = pl.multiple_of(base + k * C, 8)
      cp1 = pltpu.async_copy(sdst_h.at[pl.ds(p0, C)], dstv, sem_a)
      cp2 = pltpu.async_copy(ssrc_h.at[pl.ds(p0, C)], srcv, sem_b)
      cp1.wait()
      cp2.wait()
      for j in range(C // 16):
        d = dstv[pl.ds(j * 16, 16)]
        loc = d - chalf
        pos = (p0 + j * 16) + iot
        valid = (loc >= 0) & (loc < HALF) & (pos < limit)
        sidxv[pl.ds(j * 16, 16)] = jnp.where(valid, loc, TRASH)
        gidxv[pl.ds(j * 16, 16)] = jnp.where(valid, d, 0)
      cpa = pltpu.async_copy(a_h.at[gidxv], arow, sem_a)
      cpb = pltpu.async_copy(b_h.at[srcv], brow, sem_b)
      cpa.wait()
      cpb.wait()

      def mrow(r, cc):
        for f in range(4):
          sl = pl.ds(f * 16, 16)
          y = arow[r, sl] + brow[r, sl]
          m = jnp.where(y > 0.0, y, jnp.exp(y) - 1.0)
          arow[r, sl] = m * scale[f] + shift[f]
        return cc
      lax.fori_loop(0, C, mrow, 0)

      pltpu.sync_copy(arow, acc.at[sidxv], add=True)
      return carry
    lax.fori_loop(0, NCH, chunk, 0)

    plsc.subcore_barrier()

    ob = chalf + t * RPT
    pltpu.sync_copy(acc.at[pl.ds(t * RPT, RPT)], acc_out.at[pl.ds(ob, RPT)])

  return pl.kernel(body, out_type=out_type, mesh=mesh,
                   scratch_types=scratch, name="edge_pass",
                   compiler_params=pltpu.CompilerParams(
                       use_tc_tiling_on_sc=False,
                       needs_layout_passes=False))


_R = 3136
_GRID = NPAD // _R


def _elu(x):
  return jnp.where(x > 0.0, x, jnp.exp(x) - 1.0)


def _rows_spec(w):
  return pl.BlockSpec((_R, w), lambda i: (i, 0))


def _full_spec(r, w):
  return pl.BlockSpec((r, w), lambda i: (0, 0))


def _tc0_body(x_ref, w1_ref, b1_ref, w2_ref, b2_ref, wd_ref, wb_ref, bc_ref,
              h_ref, a_ref, b_ref):
  x = x_ref[...]
  h = _elu(jnp.dot(x, w1_ref[...], preferred_element_type=f32) + b1_ref[...])
  h = _elu(jnp.dot(h, w2_ref[...], preferred_element_type=f32) + b2_ref[...])
  h_ref[...] = h
  a_ref[...] = jnp.dot(h, wd_ref[...], preferred_element_type=f32) + bc_ref[...]
  b_ref[...] = jnp.dot(h, wb_ref[...], preferred_element_type=f32)


def _tc0(xpad, w1, b1, w2, b2, wd, wb, bc_):
  return pl.pallas_call(
      _tc0_body,
      grid=(_GRID,),
      in_specs=[_rows_spec(15), _full_spec(15, H), _full_spec(1, H),
                _full_spec(H, H), _full_spec(1, H), _full_spec(H, H),
                _full_spec(H, H), _full_spec(1, H)],
      out_specs=[_rows_spec(H), _rows_spec(H), _rows_spec(H)],
      out_shape=[jax.ShapeDtypeStruct((NPAD, H), f32)] * 3,
  )(xpad, w1, b1, w2, b2, wd, wb, bc_)


def _tcmid_body(acc_ref, deg_ref, h_ref, wd_ref, wb_ref, bc_ref,
                hn_ref, a_ref, b_ref):
  dinv = 1.0 / jnp.maximum(deg_ref[...][:, 0:1], 1.0)
  hn = acc_ref[...] * dinv + h_ref[...]
  hn_ref[...] = hn
  a_ref[...] = jnp.dot(hn, wd_ref[...], preferred_element_type=f32) + bc_ref[...]
  b_ref[...] = jnp.dot(hn, wb_ref[...], preferred_element_type=f32)


def _tcmid(acc, deg, h, wd, wb, bc_):
  return pl.pallas_call(
      _tcmid_body,
      grid=(_GRID,),
      in_specs=[_rows_spec(H), _rows_spec(H), _rows_spec(H),
                _full_spec(H, H), _full_spec(H, H), _full_spec(1, H)],
      out_specs=[_rows_spec(H), _rows_spec(H), _rows_spec(H)],
      out_shape=[jax.ShapeDtypeStruct((NPAD, H), f32)] * 3,
  )(acc, deg, h, wd, wb, bc_)


def _tc3_body(acc_ref, deg_ref, h_ref, wo1_ref, bo1_ref, wo2_ref, bo2_ref,
              wo3_ref, bo3_ref, o_ref):
  dinv = 1.0 / jnp.maximum(deg_ref[...][:, 0:1], 1.0)
  hn = acc_ref[...] * dinv + h_ref[...]
  o = _elu(jnp.dot(hn, wo1_ref[...], preferred_element_type=f32) + bo1_ref[...])
  o = _elu(jnp.dot(o, wo2_ref[...], preferred_element_type=f32) + bo2_ref[...])
  o_ref[...] = jnp.dot(o, wo3_ref[...], preferred_element_type=f32) + bo3_ref[...]


def _tc3(acc, deg, h, wo1, bo1, wo2, bo2, wo3, bo3):
  return pl.pallas_call(
      _tc3_body,
      grid=(_GRID,),
      in_specs=[_rows_spec(H), _rows_spec(H), _rows_spec(H),
                _full_spec(H, 32), _full_spec(1, 32), _full_spec(32, 16),
                _full_spec(1, 16), _full_spec(16, 8), _full_spec(1, 8)],
      out_specs=[_rows_spec(8)],
      out_shape=[jax.ShapeDtypeStruct((NPAD, 8), f32)],
  )(acc, deg, h, wo1, bo1, wo2, bo2, wo3, bo3)[0]


def kernel(x, edge_index, batch, W_lc1, b_lc1, W_lc2, b_lc2, Wc, bc, gc,
           betac, rmc, rvc, Wo1, bo1, Wo2, bo2, Wo3, bo3):
  ssrc = jnp.pad(edge_index[0], (0, EPAD - E))
  sdst = jnp.pad(edge_index[1], (0, EPAD - E), constant_values=FILLDST)

  xpad = jnp.pad(x, ((0, NPAD - N), (0, 0)))
  scale = gc / jnp.sqrt(rvc + 1e-5)
  shift = betac - rmc * scale
  wd = Wc[:, :H, :] - Wc[:, H:, :]
  wb = Wc[:, H:, :]

  h, a, bm = _tc0(xpad, W_lc1, b_lc1.reshape(1, H), W_lc2,
                  b_lc2.reshape(1, H), wd[0], wb[0], bc[0].reshape(1, H))
  acc, deg = _make_edge_pass(True)(a, bm, sdst, ssrc, scale[0], shift[0])
  for i in (1, 2):
    h, a, bm = _tcmid(acc, deg, h, wd[i], wb[i], bc[i].reshape(1, H))
    acc = _make_edge_pass(False)(a, bm, sdst, ssrc, scale[i], shift[i])[0]

  o = _tc3(acc, deg, h, Wo1, bo1.reshape(1, 32), Wo2, bo2.reshape(1, 16),
           Wo3, bo3.reshape(1, 8))
  return o[:N], batch


# speedup vs baseline: 5.2333x; 5.2333x over previous
import functools

import jax
import jax.numpy as jnp
from jax import lax
from jax.experimental import pallas as pl
from jax.experimental.pallas import tpu as pltpu
from jax.experimental.pallas import tpu_sc as plsc

N = 50000
E = 800000
H = 64

HALF = 25088
NPAD = 2 * HALF
RPT = HALF // 16
TRASH = HALF
ACCR = HALF + 8
C = 128
EP = E + 2048
FILLDST = 1 << 30

f32 = jnp.float32
i32 = jnp.int32


@functools.lru_cache(maxsize=None)
def _make_edge_pass(with_deg):
  mesh = plsc.VectorSubcoreMesh(
      core_axis_name="c", subcore_axis_name="s", num_cores=2, num_subcores=16)

  out_type = [jax.ShapeDtypeStruct((NPAD, H), f32)]
  scratch = [
      pltpu.VMEM((16,), i32),
      pltpu.VMEM((H,), f32),
      pltpu.VMEM((H,), f32),
      pltpu.VMEM((C,), i32),
      pltpu.VMEM((C,), i32),
      pltpu.VMEM((C,), i32),
      pltpu.VMEM((C,), i32),
      pltpu.VMEM((C, H), f32),
      pltpu.VMEM((C, H), f32),
      pltpu.VMEM_SHARED((ACCR, H), f32),
      pltpu.SemaphoreType.DMA,
      pltpu.SemaphoreType.DMA,
  ]
  if with_deg:
    out_type.append(jax.ShapeDtypeStruct((NPAD, H), f32))

  def body(a_h, b_h, sdst_h, ssrc_h, qinfo_h, sc_h, sh_h, *refs):
    if with_deg:
      (acc_out, deg_out, qv, scv, shv, dstv, srcv, gidxv, sidxv,
       arow, brow, acc, sem_a, sem_b) = refs
    else:
      (acc_out, qv, scv, shv, dstv, srcv, gidxv, sidxv,
       arow, brow, acc, sem_a, sem_b) = refs
      deg_out = None

    c = lax.axis_index("c")
    t = lax.axis_index("s")

    pltpu.sync_copy(qinfo_h, qv)
    iot16 = lax.iota(i32, 16)
    vq = qv[pl.ds(0, 16)]
    k0 = jnp.sum(jnp.where(iot16 == 0, vq, 0))
    q1 = jnp.sum(jnp.where(iot16 == 1, vq, 0))
    k1 = jnp.sum(jnp.where(iot16 == 2, vq, 0))

    pltpu.sync_copy(sc_h, scv)
    pltpu.sync_copy(sh_h, shv)
    scale = [scv[pl.ds(f * 16, 16)] for f in range(4)]
    shift = [shv[pl.ds(f * 16, 16)] for f in range(4)]

    def fill_arow(val):
      def frow(r, carry):
        for f in range(4):
          arow[r, pl.ds(f * 16, 16)] = jnp.full((16,), val, f32)
        return carry
      lax.fori_loop(0, C, frow, 0)

    zbase = t * RPT

    def zero_acc():
      for i in range(12):
        pltpu.sync_copy(arow, acc.at[pl.ds(zbase + i * C, C)])
      pltpu.sync_copy(arow.at[pl.ds(0, 32)],
                      acc.at[pl.ds(zbase + 12 * C, 32)])

    start = c * q1
    nn = (1 - c) * k0 + c * k1
    pt = ((nn + 15) // 16 + 7) // 8 * 8
    base = start + t * pt
    limit = jnp.minimum(start + nn, base + pt)
    nch = (pt + C - 1) // C
    chalf = c * HALF
    iot = lax.iota(i32, 16)

    if with_deg:
      fill_arow(0.0)
      zero_acc()
      plsc.subcore_barrier()
      fill_arow(1.0)

      def dchunk(k, carry):
        p0 = pl.multiple_of(base + k * C, 8)
        pltpu.async_copy(sdst_h.at[pl.ds(p0, C)], dstv, sem_a).wait()
        for j in range(C // 16):
          d = dstv[pl.ds(j * 16, 16)]
          loc = d - chalf
          pos = (p0 + j * 16) + iot
          valid = (loc >= 0) & (loc < HALF) & (pos < limit)
          sidxv[pl.ds(j * 16, 16)] = jnp.where(valid, loc, TRASH)
        pltpu.sync_copy(arow, acc.at[sidxv], add=True)
        return carry
      lax.fori_loop(0, nch, dchunk, 0)

      plsc.subcore_barrier()
      pltpu.sync_copy(acc.at[pl.ds(t * RPT, RPT)],
                      deg_out.at[pl.ds(chalf + t * RPT, RPT)])

    fill_arow(0.0)
    zero_acc()
    plsc.subcore_barrier()

    def chunk(k, carry):
      p0 = pl.multiple_of(base + k * C, 8)
      cp1 = pltpu.async_copy(sdst_h.at[pl.ds(p0, C)], dstv, sem_a)
      cp2 = pltpu.async_copy(ssrc_h.at[pl.ds(p0, C)], srcv, sem_b)
      cp1.wait()
      cp2.wait()
      for j in range(C // 16):
        d = dstv[pl.ds(j * 16, 16)]
        loc = d - chalf
        pos = (p0 + j * 16) + iot
        valid = (loc >= 0) & (loc < HALF) & (pos < limit)
        sidxv[pl.ds(j * 16, 16)] = jnp.where(valid, loc, TRASH)
        gidxv[pl.ds(j * 16, 16)] = jnp.where(valid, d, 0)
      cpa = pltpu.async_copy(a_h.at[gidxv], arow, sem_a)
      cpb = pltpu.async_copy(b_h.at[srcv], brow, sem_b)
      cpa.wait()
      cpb.wait()

      def mrow(r, cc):
        for f in range(4):
          sl = pl.ds(f * 16, 16)
          y = arow[r, sl] + brow[r, sl]
          m = jnp.where(y > 0.0, y, jnp.exp(y) - 1.0)
          arow[r, sl] = m * scale[f] + shift[f]
        return cc
      lax.fori_loop(0, C, mrow, 0)

      pltpu.sync_copy(arow, acc.at[sidxv], add=True)
      return carry
    lax.fori_loop(0, nch, chunk, 0)

    plsc.subcore_barrier()

    ob = chalf + t * RPT
    pltpu.sync_copy(acc.at[pl.ds(t * RPT, RPT)], acc_out.at[pl.ds(ob, RPT)])

  return pl.kernel(body, out_type=out_type, mesh=mesh,
                   scratch_types=scratch, name="edge_pass",
                   compiler_params=pltpu.CompilerParams(
                       use_tc_tiling_on_sc=False,
                       needs_layout_passes=False))


_R = 3136
_GRID = NPAD // _R


def _elu(x):
  return jnp.where(x > 0.0, x, jnp.exp(x) - 1.0)


def _rows_spec(w):
  return pl.BlockSpec((_R, w), lambda i: (i, 0))


def _full_spec(r, w):
  return pl.BlockSpec((r, w), lambda i: (0, 0))


def _tc0_body(x_ref, w1_ref, b1_ref, w2_ref, b2_ref, wd_ref, wb_ref, bc_ref,
              h_ref, a_ref, b_ref):
  x = x_ref[...]
  h = _elu(jnp.dot(x, w1_ref[...], preferred_element_type=f32) + b1_ref[...])
  h = _elu(jnp.dot(h, w2_ref[...], preferred_element_type=f32) + b2_ref[...])
  h_ref[...] = h
  a_ref[...] = jnp.dot(h, wd_ref[...], preferred_element_type=f32) + bc_ref[...]
  b_ref[...] = jnp.dot(h, wb_ref[...], preferred_element_type=f32)


def _tc0(xpad, w1, b1, w2, b2, wd, wb, bc_):
  return pl.pallas_call(
      _tc0_body,
      grid=(_GRID,),
      in_specs=[_rows_spec(15), _full_spec(15, H), _full_spec(1, H),
                _full_spec(H, H), _full_spec(1, H), _full_spec(H, H),
                _full_spec(H, H), _full_spec(1, H)],
      out_specs=[_rows_spec(H), _rows_spec(H), _rows_spec(H)],
      out_shape=[jax.ShapeDtypeStruct((NPAD, H), f32)] * 3,
  )(xpad, w1, b1, w2, b2, wd, wb, bc_)


def _tcmid_body(acc_ref, deg_ref, h_ref, wd_ref, wb_ref, bc_ref,
                hn_ref, a_ref, b_ref):
  dinv = 1.0 / jnp.maximum(deg_ref[...][:, 0:1], 1.0)
  hn = acc_ref[...] * dinv + h_ref[...]
  hn_ref[...] = hn
  a_ref[...] = jnp.dot(hn, wd_ref[...], preferred_element_type=f32) + bc_ref[...]
  b_ref[...] = jnp.dot(hn, wb_ref[...], preferred_element_type=f32)


def _tcmid(acc, deg, h, wd, wb, bc_):
  return pl.pallas_call(
      _tcmid_body,
      grid=(_GRID,),
      in_specs=[_rows_spec(H), _rows_spec(H), _rows_spec(H),
                _full_spec(H, H), _full_spec(H, H), _full_spec(1, H)],
      out_specs=[_rows_spec(H), _rows_spec(H), _rows_spec(H)],
      out_shape=[jax.ShapeDtypeStruct((NPAD, H), f32)] * 3,
  )(acc, deg, h, wd, wb, bc_)


def _tc3_body(acc_ref, deg_ref, h_ref, wo1_ref, bo1_ref, wo2_ref, bo2_ref,
              wo3_ref, bo3_ref, o_ref):
  dinv = 1.0 / jnp.maximum(deg_ref[...][:, 0:1], 1.0)
  hn = acc_ref[...] * dinv + h_ref[...]
  o = _elu(jnp.dot(hn, wo1_ref[...], preferred_element_type=f32) + bo1_ref[...])
  o = _elu(jnp.dot(o, wo2_ref[...], preferred_element_type=f32) + bo2_ref[...])
  o_ref[...] = jnp.dot(o, wo3_ref[...], preferred_element_type=f32) + bo3_ref[...]


def _tc3(acc, deg, h, wo1, bo1, wo2, bo2, wo3, bo3):
  return pl.pallas_call(
      _tc3_body,
      grid=(_GRID,),
      in_specs=[_rows_spec(H), _rows_spec(H), _rows_spec(H),
                _full_spec(H, 32), _full_spec(1, 32), _full_spec(32, 16),
                _full_spec(1, 16), _full_spec(16, 8), _full_spec(1, 8)],
      out_specs=[_rows_spec(8)],
      out_shape=[jax.ShapeDtypeStruct((NPAD, 8), f32)],
  )(acc, deg, h, wo1, bo1, wo2, bo2, wo3, bo3)[0]


def kernel(x, edge_index, batch, W_lc1, b_lc1, W_lc2, b_lc2, Wc, bc, gc,
           betac, rmc, rvc, Wo1, bo1, Wo2, bo2, Wo3, bo3):
  src = edge_index[0]
  dst = edge_index[1]

  m0 = dst < HALF
  cs = jnp.cumsum(m0.astype(i32))
  k0 = cs[-1]
  q1 = (k0 + 127) // 128 * 128
  ar = jnp.arange(E, dtype=i32)
  pos = jnp.where(m0, cs - 1, q1 + (ar - cs))
  sdst = jnp.full((EP,), FILLDST, i32).at[pos].set(dst, unique_indices=True)
  ssrc = jnp.zeros((EP,), i32).at[pos].set(src, unique_indices=True)
  qinfo = (jnp.zeros((16,), i32)
           .at[0].set(k0).at[1].set(q1).at[2].set(E - k0))

  xpad = jnp.pad(x, ((0, NPAD - N), (0, 0)))
  scale = gc / jnp.sqrt(rvc + 1e-5)
  shift = betac - rmc * scale
  wd = Wc[:, :H, :] - Wc[:, H:, :]
  wb = Wc[:, H:, :]

  h, a, bm = _tc0(xpad, W_lc1, b_lc1.reshape(1, H), W_lc2,
                  b_lc2.reshape(1, H), wd[0], wb[0], bc[0].reshape(1, H))
  acc, deg = _make_edge_pass(True)(a, bm, sdst, ssrc, qinfo,
                                   scale[0], shift[0])
  for i in (1, 2):
    h, a, bm = _tcmid(acc, deg, h, wd[i], wb[i], bc[i].reshape(1, H))
    acc = _make_edge_pass(False)(a, bm, sdst, ssrc, qinfo,
                                 scale[i], shift[i])[0]

  o = _tc3(acc, deg, h, Wo1, bo1.reshape(1, 32), Wo2, bo2.reshape(1, 16),
           Wo3, bo3.reshape(1, 8))
  return o[:N], batch


# speedup vs baseline: 14.3575x; 2.7435x over previous
import functools

import jax
import jax.numpy as jnp
from jax import lax
from jax.experimental import pallas as pl
from jax.experimental.pallas import tpu as pltpu
from jax.experimental.pallas import tpu_sc as plsc

N = 50000
E = 800000
H = 64

HALF = 25088
NPAD = 2 * HALF
RPT = HALF // 16
TRASH = HALF
ACCR = HALF + 8
C = 128
ET = E // 32
PNCH = (ET + C - 1) // C
ETP = 25104
RW = 32 * ETP
FILLDST = 1 << 30

f32 = jnp.float32
i32 = jnp.int32

_SC_PARAMS = pltpu.CompilerParams(
    use_tc_tiling_on_sc=False, needs_layout_passes=False)


def _mesh():
  return plsc.VectorSubcoreMesh(
      core_axis_name="c", subcore_axis_name="s", num_cores=2, num_subcores=16)


def _lane0(qv, iot16):
  return jnp.sum(jnp.where(iot16 == 0, qv[pl.ds(0, 16)], 0))


@functools.lru_cache(maxsize=None)
def _make_partition():
  out_type = [
      jax.ShapeDtypeStruct((2, RW), i32),
      jax.ShapeDtypeStruct((2, RW), i32),
      jax.ShapeDtypeStruct((64, 16), i32),
  ]
  scratch = [
      pltpu.VMEM((C,), i32),
      pltpu.VMEM((C,), i32),
      pltpu.VMEM((ETP,), i32),
      pltpu.VMEM((ETP,), i32),
      pltpu.VMEM((ETP,), i32),
      pltpu.VMEM((ETP,), i32),
      pltpu.VMEM((16,), i32),
      pltpu.SemaphoreType.DMA,
      pltpu.SemaphoreType.DMA,
  ]

  def body(dst_h, src_h, sdst2, ssrc2, qcnt, dstv, srcv,
           ob0d, ob0s, ob1d, ob1s, qbuf, sem_a, sem_b):
    c = lax.axis_index("c")
    t = lax.axis_index("s")
    w = c * 16 + t
    base = w * ET
    iot = lax.iota(i32, 16)

    def chunk(k, carry):
      n0, n1 = carry
      p0 = pl.multiple_of(base + k * C, 8)
      cp1 = pltpu.async_copy(dst_h.at[pl.ds(p0, C)], dstv, sem_a)
      cp2 = pltpu.async_copy(src_h.at[pl.ds(p0, C)], srcv, sem_b)
      cp1.wait()
      cp2.wait()
      for j in range(C // 16):
        d = dstv[pl.ds(j * 16, 16)]
        s = srcv[pl.ds(j * 16, 16)]
        lpos = (k * C + j * 16) + iot
        vv = lpos < ET
        m0 = vv & (d < HALF)
        m1 = vv & (d >= HALF)
        c0 = plsc.cumsum(m0.astype(i32))
        c1 = plsc.cumsum(m1.astype(i32))
        i0 = (n0 - 1) + c0
        i1 = (n1 - 1) + c1
        plsc.store_scatter(ob0d, [i0], d, mask=m0)
        plsc.store_scatter(ob0s, [i0], s, mask=m0)
        plsc.store_scatter(ob1d, [i1], d, mask=m1)
        plsc.store_scatter(ob1s, [i1], s, mask=m1)
        n0 = n0 + jnp.sum(m0.astype(i32))
        n1 = n1 + jnp.sum(m1.astype(i32))
      return (n0, n1)

    n0, n1 = lax.fori_loop(0, PNCH, chunk, (jnp.array(0, i32),
                                            jnp.array(0, i32)))

    rbase = w * ETP
    pltpu.sync_copy(ob0d, sdst2.at[0, pl.ds(rbase, ETP)])
    pltpu.sync_copy(ob0s, ssrc2.at[0, pl.ds(rbase, ETP)])
    pltpu.sync_copy(ob1d, sdst2.at[1, pl.ds(rbase, ETP)])
    pltpu.sync_copy(ob1s, ssrc2.at[1, pl.ds(rbase, ETP)])

    qbuf[pl.ds(0, 16)] = jnp.where(iot == 0, n0, 0)
    pltpu.sync_copy(qbuf, qcnt.at[w])
    qbuf[pl.ds(0, 16)] = jnp.where(iot == 0, n1, 0)
    pltpu.sync_copy(qbuf, qcnt.at[32 + w])

  return pl.kernel(body, out_type=out_type, mesh=_mesh(),
                   scratch_types=scratch, name="edge_partition",
                   compiler_params=_SC_PARAMS)


@functools.lru_cache(maxsize=None)
def _make_edge_pass(with_deg):
  out_type = [jax.ShapeDtypeStruct((NPAD, H), f32)]
  scratch = [
      pltpu.VMEM((16,), i32),
      pltpu.VMEM((H,), f32),
      pltpu.VMEM((H,), f32),
      pltpu.VMEM((C,), i32),
      pltpu.VMEM((C,), i32),
      pltpu.VMEM((C,), i32),
      pltpu.VMEM((C,), i32),
      pltpu.VMEM((C, H), f32),
      pltpu.VMEM((C, H), f32),
      pltpu.VMEM_SHARED((ACCR, H), f32),
      pltpu.SemaphoreType.DMA,
      pltpu.SemaphoreType.DMA,
  ]
  if with_deg:
    out_type.append(jax.ShapeDtypeStruct((NPAD, H), f32))

  def body(a_h, b_h, sdst2, ssrc2, qcnt, sc_h, sh_h, *refs):
    if with_deg:
      (acc_out, deg_out, qv, scv, shv, dstv, srcv, gidxv, sidxv,
       arow, brow, acc, sem_a, sem_b) = refs
    else:
      (acc_out, qv, scv, shv, dstv, srcv, gidxv, sidxv,
       arow, brow, acc, sem_a, sem_b) = refs
      deg_out = None

    c = lax.axis_index("c")
    t = lax.axis_index("s")
    iot = lax.iota(i32, 16)
    chalf = c * HALF

    pltpu.sync_copy(sc_h, scv)
    pltpu.sync_copy(sh_h, shv)
    scale = [scv[pl.ds(f * 16, 16)] for f in range(4)]
    shift = [shv[pl.ds(f * 16, 16)] for f in range(4)]

    regs = []
    for rr in range(2):
      reg = 2 * t + rr
      pltpu.sync_copy(qcnt.at[c * 32 + reg], qv)
      cnt = _lane0(qv, iot)
      regs.append((reg * ETP, cnt))

    def fill_arow(val):
      def frow(r, carry):
        for f in range(4):
          arow[r, pl.ds(f * 16, 16)] = jnp.full((16,), val, f32)
        return carry
      lax.fori_loop(0, C, frow, 0)

    zbase = t * RPT

    def zero_acc():
      for i in range(12):
        pltpu.sync_copy(arow, acc.at[pl.ds(zbase + i * C, C)])
      pltpu.sync_copy(arow.at[pl.ds(0, 32)],
                      acc.at[pl.ds(zbase + 12 * C, 32)])

    def make_sidx(rbase, cnt, k):
      p0 = pl.multiple_of(rbase + k * C, 8)
      cp1 = pltpu.async_copy(sdst2.at[c, pl.ds(p0, C)], dstv, sem_a)
      cp2 = pltpu.async_copy(ssrc2.at[c, pl.ds(p0, C)], srcv, sem_b)
      cp1.wait()
      cp2.wait()
      for j in range(C // 16):
        d = dstv[pl.ds(j * 16, 16)]
        s = srcv[pl.ds(j * 16, 16)]
        lpos = (k * C + j * 16) + iot
        valid = lpos < cnt
        sidxv[pl.ds(j * 16, 16)] = jnp.where(valid, d - chalf, TRASH)
        gidxv[pl.ds(j * 16, 16)] = jnp.where(valid, d, 0)
        srcv[pl.ds(j * 16, 16)] = jnp.where(valid, s, 0)

    if with_deg:
      fill_arow(0.0)
      zero_acc()
      plsc.subcore_barrier()
      fill_arow(1.0)

      for rbase, cnt in regs:
        def dchunk(k, carry, rbase=rbase, cnt=cnt):
          make_sidx(rbase, cnt, k)
          pltpu.sync_copy(arow, acc.at[sidxv], add=True)
          return carry
        lax.fori_loop(0, (cnt + C - 1) // C, dchunk, 0)

      plsc.subcore_barrier()
      pltpu.sync_copy(acc.at[pl.ds(t * RPT, RPT)],
                      deg_out.at[pl.ds(chalf + t * RPT, RPT)])

    fill_arow(0.0)
    zero_acc()
    plsc.subcore_barrier()

    for rbase, cnt in regs:
      def chunk(k, carry, rbase=rbase, cnt=cnt):
        make_sidx(rbase, cnt, k)
        cpa = pltpu.async_copy(a_h.at[gidxv], arow, sem_a)
        cpb = pltpu.async_copy(b_h.at[srcv], brow, sem_b)
        cpa.wait()
        cpb.wait()

        def mrow(r, cc):
          for f in range(4):
            sl = pl.ds(f * 16, 16)
            y = arow[r, sl] + brow[r, sl]
            m = jnp.where(y > 0.0, y, jnp.exp(y) - 1.0)
            arow[r, sl] = m * scale[f] + shift[f]
          return cc
        lax.fori_loop(0, C, mrow, 0)

        pltpu.sync_copy(arow, acc.at[sidxv], add=True)
        return carry
      lax.fori_loop(0, (cnt + C - 1) // C, chunk, 0)

    plsc.subcore_barrier()

    pltpu.sync_copy(acc.at[pl.ds(t * RPT, RPT)],
                    acc_out.at[pl.ds(chalf + t * RPT, RPT)])

  return pl.kernel(body, out_type=out_type, mesh=_mesh(),
                   scratch_types=scratch, name="edge_pass",
                   compiler_params=_SC_PARAMS)


_R = 3136
_GRID = NPAD // _R


def _elu(x):
  return jnp.where(x > 0.0, x, jnp.exp(x) - 1.0)


def _rows_spec(w):
  return pl.BlockSpec((_R, w), lambda i: (i, 0))


def _full_spec(r, w):
  return pl.BlockSpec((r, w), lambda i: (0, 0))


def _tc0_body(x_ref, w1_ref, b1_ref, w2_ref, b2_ref, wd_ref, wb_ref, bc_ref,
              h_ref, a_ref, b_ref):
  x = x_ref[...]
  h = _elu(jnp.dot(x, w1_ref[...], preferred_element_type=f32) + b1_ref[...])
  h = _elu(jnp.dot(h, w2_ref[...], preferred_element_type=f32) + b2_ref[...])
  h_ref[...] = h
  a_ref[...] = jnp.dot(h, wd_ref[...], preferred_element_type=f32) + bc_ref[...]
  b_ref[...] = jnp.dot(h, wb_ref[...], preferred_element_type=f32)


def _tc0(xpad, w1, b1, w2, b2, wd, wb, bc_):
  return pl.pallas_call(
      _tc0_body,
      grid=(_GRID,),
      in_specs=[_rows_spec(15), _full_spec(15, H), _full_spec(1, H),
                _full_spec(H, H), _full_spec(1, H), _full_spec(H, H),
                _full_spec(H, H), _full_spec(1, H)],
      out_specs=[_rows_spec(H), _rows_spec(H), _rows_spec(H)],
      out_shape=[jax.ShapeDtypeStruct((NPAD, H), f32)] * 3,
  )(xpad, w1, b1, w2, b2, wd, wb, bc_)


def _tcmid_body(acc_ref, deg_ref, h_ref, wd_ref, wb_ref, bc_ref,
                hn_ref, a_ref, b_ref):
  dinv = 1.0 / jnp.maximum(deg_ref[...][:, 0:1], 1.0)
  hn = acc_ref[...] * dinv + h_ref[...]
  hn_ref[...] = hn
  a_ref[...] = jnp.dot(hn, wd_ref[...], preferred_element_type=f32) + bc_ref[...]
  b_ref[...] = jnp.dot(hn, wb_ref[...], preferred_element_type=f32)


def _tcmid(acc, deg, h, wd, wb, bc_):
  return pl.pallas_call(
      _tcmid_body,
      grid=(_GRID,),
      in_specs=[_rows_spec(H), _rows_spec(H), _rows_spec(H),
                _full_spec(H, H), _full_spec(H, H), _full_spec(1, H)],
      out_specs=[_rows_spec(H), _rows_spec(H), _rows_spec(H)],
      out_shape=[jax.ShapeDtypeStruct((NPAD, H), f32)] * 3,
  )(acc, deg, h, wd, wb, bc_)


def _tc3_body(acc_ref, deg_ref, h_ref, wo1_ref, bo1_ref, wo2_ref, bo2_ref,
              wo3_ref, bo3_ref, o_ref):
  dinv = 1.0 / jnp.maximum(deg_ref[...][:, 0:1], 1.0)
  hn = acc_ref[...] * dinv + h_ref[...]
  o = _elu(jnp.dot(hn, wo1_ref[...], preferred_element_type=f32) + bo1_ref[...])
  o = _elu(jnp.dot(o, wo2_ref[...], preferred_element_type=f32) + bo2_ref[...])
  o_ref[...] = jnp.dot(o, wo3_ref[...], preferred_element_type=f32) + bo3_ref[...]


def _tc3(acc, deg, h, wo1, bo1, wo2, bo2, wo3, bo3):
  return pl.pallas_call(
      _tc3_body,
      grid=(_GRID,),
      in_specs=[_rows_spec(H), _rows_spec(H), _rows_spec(H),
                _full_spec(H, 32), _full_spec(1, 32), _full_spec(32, 16),
                _full_spec(1, 16), _full_spec(16, 8), _full_spec(1, 8)],
      out_specs=[_rows_spec(8)],
      out_shape=[jax.ShapeDtypeStruct((NPAD, 8), f32)],
  )(acc, deg, h, wo1, bo1, wo2, bo2, wo3, bo3)[0]


def kernel(x, edge_index, batch, W_lc1, b_lc1, W_lc2, b_lc2, Wc, bc, gc,
           betac, rmc, rvc, Wo1, bo1, Wo2, bo2, Wo3, bo3):
  dstp = jnp.pad(edge_index[1], (0, 128), constant_values=FILLDST)
  srcp = jnp.pad(edge_index[0], (0, 128))
  sdst2, ssrc2, qcnt = _make_partition()(dstp, srcp)

  xpad = jnp.pad(x, ((0, NPAD - N), (0, 0)))
  scale = gc / jnp.sqrt(rvc + 1e-5)
  shift = betac - rmc * scale
  wd = Wc[:, :H, :] - Wc[:, H:, :]
  wb = Wc[:, H:, :]

  h, a, bm = _tc0(xpad, W_lc1, b_lc1.reshape(1, H), W_lc2,
                  b_lc2.reshape(1, H), wd[0], wb[0], bc[0].reshape(1, H))
  acc, deg = _make_edge_pass(True)(a, bm, sdst2, ssrc2, qcnt,
                                   scale[0], shift[0])
  for i in (1, 2):
    h, a, bm = _tcmid(acc, deg, h, wd[i], wb[i], bc[i].reshape(1, H))
    acc = _make_edge_pass(False)(a, bm, sdst2, ssrc2, qcnt,
                                 scale[i], shift[i])[0]

  o = _tc3(acc, deg, h, Wo1, bo1.reshape(1, 32), Wo2, bo2.reshape(1, 16),
           Wo3, bo3.reshape(1, 8))
  return o[:N], batch


# speedup vs baseline: 19.3385x; 1.3469x over previous
import functools

import jax
import jax.numpy as jnp
from jax import lax
from jax.experimental import pallas as pl
from jax.experimental.pallas import tpu as pltpu
from jax.experimental.pallas import tpu_sc as plsc

N = 50000
E = 800000
H = 64

HALF = 25088
NPAD = 2 * HALF
RPT = HALF // 16
TRASH = HALF
ACCR = HALF + 8
C = 64
ET = E // 32
PNCH = (ET + C - 1) // C
ETP = 25104
RW = 32 * ETP
FILLDST = 1 << 30

f32 = jnp.float32
i32 = jnp.int32

_SC_PARAMS = pltpu.CompilerParams(
    use_tc_tiling_on_sc=False, needs_layout_passes=False)


def _mesh():
  return plsc.VectorSubcoreMesh(
      core_axis_name="c", subcore_axis_name="s", num_cores=2, num_subcores=16)


def _lane0(qv, iot16):
  return jnp.sum(jnp.where(iot16 == 0, qv[pl.ds(0, 16)], 0))


@functools.lru_cache(maxsize=None)
def _make_partition():
  out_type = [
      jax.ShapeDtypeStruct((2, RW), i32),
      jax.ShapeDtypeStruct((2, RW), i32),
      jax.ShapeDtypeStruct((64, 16), i32),
  ]
  scratch = [
      pltpu.VMEM((C,), i32),
      pltpu.VMEM((C,), i32),
      pltpu.VMEM((ETP,), i32),
      pltpu.VMEM((ETP,), i32),
      pltpu.VMEM((ETP,), i32),
      pltpu.VMEM((ETP,), i32),
      pltpu.VMEM((16,), i32),
      pltpu.SemaphoreType.DMA,
      pltpu.SemaphoreType.DMA,
  ]

  def body(dst_h, src_h, sdst2, ssrc2, qcnt, dstv, srcv,
           ob0d, ob0s, ob1d, ob1s, qbuf, sem_a, sem_b):
    c = lax.axis_index("c")
    t = lax.axis_index("s")
    w = c * 16 + t
    base = w * ET
    iot = lax.iota(i32, 16)

    def chunk(k, carry):
      n0, n1 = carry
      p0 = pl.multiple_of(base + k * C, 8)
      cp1 = pltpu.async_copy(dst_h.at[pl.ds(p0, C)], dstv, sem_a)
      cp2 = pltpu.async_copy(src_h.at[pl.ds(p0, C)], srcv, sem_b)
      cp1.wait()
      cp2.wait()
      for j in range(C // 16):
        d = dstv[pl.ds(j * 16, 16)]
        s = srcv[pl.ds(j * 16, 16)]
        lpos = (k * C + j * 16) + iot
        vv = lpos < ET
        m0 = vv & (d < HALF)
        m1 = vv & (d >= HALF)
        c0 = plsc.cumsum(m0.astype(i32))
        c1 = plsc.cumsum(m1.astype(i32))
        i0 = (n0 - 1) + c0
        i1 = (n1 - 1) + c1
        plsc.store_scatter(ob0d, [i0], d, mask=m0)
        plsc.store_scatter(ob0s, [i0], s, mask=m0)
        plsc.store_scatter(ob1d, [i1], d, mask=m1)
        plsc.store_scatter(ob1s, [i1], s, mask=m1)
        n0 = n0 + jnp.sum(m0.astype(i32))
        n1 = n1 + jnp.sum(m1.astype(i32))
      return (n0, n1)

    n0, n1 = lax.fori_loop(0, PNCH, chunk, (jnp.array(0, i32),
                                            jnp.array(0, i32)))

    rbase = w * ETP
    pltpu.sync_copy(ob0d, sdst2.at[0, pl.ds(rbase, ETP)])
    pltpu.sync_copy(ob0s, ssrc2.at[0, pl.ds(rbase, ETP)])
    pltpu.sync_copy(ob1d, sdst2.at[1, pl.ds(rbase, ETP)])
    pltpu.sync_copy(ob1s, ssrc2.at[1, pl.ds(rbase, ETP)])

    qbuf[pl.ds(0, 16)] = jnp.where(iot == 0, n0, 0)
    pltpu.sync_copy(qbuf, qcnt.at[w])
    qbuf[pl.ds(0, 16)] = jnp.where(iot == 0, n1, 0)
    pltpu.sync_copy(qbuf, qcnt.at[32 + w])

  return pl.kernel(body, out_type=out_type, mesh=_mesh(),
                   scratch_types=scratch, name="edge_partition",
                   compiler_params=_SC_PARAMS)


@functools.lru_cache(maxsize=None)
def _make_edge_pass(with_deg):
  out_type = [jax.ShapeDtypeStruct((NPAD, H), f32)]
  scratch = [
      pltpu.VMEM((16,), i32),
      pltpu.VMEM((H,), f32),
      pltpu.VMEM((H,), f32),
      [pltpu.VMEM((C,), i32) for _ in range(2)],
      [pltpu.VMEM((C,), i32) for _ in range(2)],
      [pltpu.VMEM((C,), i32) for _ in range(2)],
      [pltpu.VMEM((C,), i32) for _ in range(2)],
      [pltpu.VMEM((C, H), f32) for _ in range(2)],
      [pltpu.VMEM((C, H), f32) for _ in range(2)],
      pltpu.VMEM_SHARED((ACCR, H), f32),
      pltpu.SemaphoreType.DMA,
      pltpu.SemaphoreType.DMA,
      [pltpu.SemaphoreType.DMA for _ in range(2)],
      [pltpu.SemaphoreType.DMA for _ in range(2)],
  ]
  if with_deg:
    out_type.append(jax.ShapeDtypeStruct((NPAD, H), f32))

  def body(a_h, b_h, sdst2, ssrc2, qcnt, sc_h, sh_h, *refs):
    if with_deg:
      (acc_out, deg_out, qv, scv, shv, dstv, srcv, gidxv, sidxv,
       arow, brow, acc, sem_i, sem_j, sem_a, sem_b) = refs
    else:
      (acc_out, qv, scv, shv, dstv, srcv, gidxv, sidxv,
       arow, brow, acc, sem_i, sem_j, sem_a, sem_b) = refs
      deg_out = None

    c = lax.axis_index("c")
    t = lax.axis_index("s")
    iot = lax.iota(i32, 16)
    chalf = c * HALF

    pltpu.sync_copy(sc_h, scv)
    pltpu.sync_copy(sh_h, shv)
    scale = [scv[pl.ds(f * 16, 16)] for f in range(4)]
    shift = [shv[pl.ds(f * 16, 16)] for f in range(4)]

    rb0 = (2 * t) * ETP
    rb1 = (2 * t + 1) * ETP
    pltpu.sync_copy(qcnt.at[c * 32 + 2 * t], qv)
    cnt0 = _lane0(qv, iot)
    pltpu.sync_copy(qcnt.at[c * 32 + 2 * t + 1], qv)
    cnt1 = _lane0(qv, iot)
    t0 = (cnt0 + C - 1) // C
    tt = t0 + (cnt1 + C - 1) // C

    def off_cnt(k):
      in0 = k < t0
      koff = jnp.where(in0, k, k - t0) * C
      addr = jnp.where(in0, rb0, rb1) + koff
      cc = jnp.where(in0, cnt0, cnt1)
      return koff, addr, cc

    def load_idx(k, b, make_gidx):
      koff, addr, cc = off_cnt(k)
      p0 = pl.multiple_of(addr, 8)
      cp1 = pltpu.async_copy(sdst2.at[c, pl.ds(p0, C)], dstv[b], sem_i)
      cp2 = pltpu.async_copy(ssrc2.at[c, pl.ds(p0, C)], srcv[b], sem_j)
      cp1.wait()
      cp2.wait()
      for j in range(C // 16):
        sl = pl.ds(j * 16, 16)
        d = dstv[b][sl]
        s = srcv[b][sl]
        valid = ((koff + j * 16) + iot) < cc
        sidxv[b][sl] = jnp.where(valid, d - chalf, TRASH)
        if make_gidx:
          gidxv[b][sl] = jnp.where(valid, d, 0)
          srcv[b][sl] = jnp.where(valid, s, 0)

    def issue_gathers(k, b):
      load_idx(k, b, True)
      pltpu.async_copy(a_h.at[gidxv[b]], arow[b], sem_a[b])
      pltpu.async_copy(b_h.at[srcv[b]], brow[b], sem_b[b])

    def fill_arow(b, val):
      def frow(r, carry):
        for f in range(4):
          arow[b][r, pl.ds(f * 16, 16)] = jnp.full((16,), val, f32)
        return carry
      lax.fori_loop(0, C, frow, 0)

    zbase = t * RPT

    def zero_acc():
      nf, rem = RPT // C, RPT % C
      for i in range(nf):
        pltpu.sync_copy(arow[0], acc.at[pl.ds(zbase + i * C, C)])
      if rem:
        pltpu.sync_copy(arow[0].at[pl.ds(0, rem)],
                        acc.at[pl.ds(zbase + nf * C, rem)])

    if with_deg:
      fill_arow(0, 0.0)
      zero_acc()
      plsc.subcore_barrier()
      fill_arow(0, 1.0)

      def dchunk(k, carry):
        load_idx(k, 0, False)
        pltpu.sync_copy(arow[0], acc.at[sidxv[0]], add=True)
        return carry
      lax.fori_loop(0, tt, dchunk, 0)

      plsc.subcore_barrier()
      pltpu.sync_copy(acc.at[pl.ds(t * RPT, RPT)],
                      deg_out.at[pl.ds(chalf + t * RPT, RPT)])

    fill_arow(0, 0.0)
    zero_acc()
    plsc.subcore_barrier()

    @pl.when(tt > 0)
    def _():
      issue_gathers(0, 0)

    @pl.when(tt > 1)
    def _():
      issue_gathers(1, 1)

    def outer(g, carry):
      for b in range(2):
        k = 2 * g + b

        @pl.when(k < tt)
        def _(b=b, k=k):
          pltpu.make_async_copy(a_h.at[gidxv[b]], arow[b], sem_a[b]).wait()
          pltpu.make_async_copy(b_h.at[srcv[b]], brow[b], sem_b[b]).wait()

          def mrow(r, cc):
            for rr in range(2):
              for f in range(4):
                sl = pl.ds(f * 16, 16)
                y = arow[b][2 * r + rr, sl] + brow[b][2 * r + rr, sl]
                m = jnp.where(y > 0.0, y, jnp.exp(y) - 1.0)
                arow[b][2 * r + rr, sl] = m * scale[f] + shift[f]
            return cc
          lax.fori_loop(0, C // 2, mrow, 0)

          pltpu.sync_copy(arow[b], acc.at[sidxv[b]], add=True)

          @pl.when(k + 2 < tt)
          def _():
            issue_gathers(k + 2, b)
      return carry
    lax.fori_loop(0, (tt + 1) // 2, outer, 0)

    plsc.subcore_barrier()

    pltpu.sync_copy(acc.at[pl.ds(t * RPT, RPT)],
                    acc_out.at[pl.ds(chalf + t * RPT, RPT)])

  return pl.kernel(body, out_type=out_type, mesh=_mesh(),
                   scratch_types=scratch, name="edge_pass",
                   compiler_params=_SC_PARAMS)


_R = 3136
_GRID = NPAD // _R


def _elu(x):
  return jnp.where(x > 0.0, x, jnp.exp(x) - 1.0)


def _rows_spec(w):
  return pl.BlockSpec((_R, w), lambda i: (i, 0))


def _full_spec(r, w):
  return pl.BlockSpec((r, w), lambda i: (0, 0))


def _tc0_body(x_ref, w1_ref, b1_ref, w2_ref, b2_ref, wd_ref, wb_ref, bc_ref,
              h_ref, a_ref, b_ref):
  x = x_ref[...]
  h = _elu(jnp.dot(x, w1_ref[...], preferred_element_type=f32) + b1_ref[...])
  h = _elu(jnp.dot(h, w2_ref[...], preferred_element_type=f32) + b2_ref[...])
  h_ref[...] = h
  a_ref[...] = jnp.dot(h, wd_ref[...], preferred_element_type=f32) + bc_ref[...]
  b_ref[...] = jnp.dot(h, wb_ref[...], preferred_element_type=f32)


def _tc0(xpad, w1, b1, w2, b2, wd, wb, bc_):
  return pl.pallas_call(
      _tc0_body,
      grid=(_GRID,),
      in_specs=[_rows_spec(15), _full_spec(15, H), _full_spec(1, H),
                _full_spec(H, H), _full_spec(1, H), _full_spec(H, H),
                _full_spec(H, H), _full_spec(1, H)],
      out_specs=[_rows_spec(H), _rows_spec(H), _rows_spec(H)],
      out_shape=[jax.ShapeDtypeStruct((NPAD, H), f32)] * 3,
  )(xpad, w1, b1, w2, b2, wd, wb, bc_)


def _tcmid_body(acc_ref, deg_ref, h_ref, wd_ref, wb_ref, bc_ref,
                hn_ref, a_ref, b_ref):
  dinv = 1.0 / jnp.maximum(deg_ref[...][:, 0:1], 1.0)
  hn = acc_ref[...] * dinv + h_ref[...]
  hn_ref[...] = hn
  a_ref[...] = jnp.dot(hn, wd_ref[...], preferred_element_type=f32) + bc_ref[...]
  b_ref[...] = jnp.dot(hn, wb_ref[...], preferred_element_type=f32)


def _tcmid(acc, deg, h, wd, wb, bc_):
  return pl.pallas_call(
      _tcmid_body,
      grid=(_GRID,),
      in_specs=[_rows_spec(H), _rows_spec(H), _rows_spec(H),
                _full_spec(H, H), _full_spec(H, H), _full_spec(1, H)],
      out_specs=[_rows_spec(H), _rows_spec(H), _rows_spec(H)],
      out_shape=[jax.ShapeDtypeStruct((NPAD, H), f32)] * 3,
  )(acc, deg, h, wd, wb, bc_)


def _tc3_body(acc_ref, deg_ref, h_ref, wo1_ref, bo1_ref, wo2_ref, bo2_ref,
              wo3_ref, bo3_ref, o_ref):
  dinv = 1.0 / jnp.maximum(deg_ref[...][:, 0:1], 1.0)
  hn = acc_ref[...] * dinv + h_ref[...]
  o = _elu(jnp.dot(hn, wo1_ref[...], preferred_element_type=f32) + bo1_ref[...])
  o = _elu(jnp.dot(o, wo2_ref[...], preferred_element_type=f32) + bo2_ref[...])
  o_ref[...] = jnp.dot(o, wo3_ref[...], preferred_element_type=f32) + bo3_ref[...]


def _tc3(acc, deg, h, wo1, bo1, wo2, bo2, wo3, bo3):
  return pl.pallas_call(
      _tc3_body,
      grid=(_GRID,),
      in_specs=[_rows_spec(H), _rows_spec(H), _rows_spec(H),
                _full_spec(H, 32), _full_spec(1, 32), _full_spec(32, 16),
                _full_spec(1, 16), _full_spec(16, 8), _full_spec(1, 8)],
      out_specs=[_rows_spec(8)],
      out_shape=[jax.ShapeDtypeStruct((NPAD, 8), f32)],
  )(acc, deg, h, wo1, bo1, wo2, bo2, wo3, bo3)[0]


def kernel(x, edge_index, batch, W_lc1, b_lc1, W_lc2, b_lc2, Wc, bc, gc,
           betac, rmc, rvc, Wo1, bo1, Wo2, bo2, Wo3, bo3):
  dstp = jnp.pad(edge_index[1], (0, 128), constant_values=FILLDST)
  srcp = jnp.pad(edge_index[0], (0, 128))
  sdst2, ssrc2, qcnt = _make_partition()(dstp, srcp)

  xpad = jnp.pad(x, ((0, NPAD - N), (0, 0)))
  scale = gc / jnp.sqrt(rvc + 1e-5)
  shift = betac - rmc * scale
  wd = Wc[:, :H, :] - Wc[:, H:, :]
  wb = Wc[:, H:, :]

  h, a, bm = _tc0(xpad, W_lc1, b_lc1.reshape(1, H), W_lc2,
                  b_lc2.reshape(1, H), wd[0], wb[0], bc[0].reshape(1, H))
  acc, deg = _make_edge_pass(True)(a, bm, sdst2, ssrc2, qcnt,
                                   scale[0], shift[0])
  for i in (1, 2):
    h, a, bm = _tcmid(acc, deg, h, wd[i], wb[i], bc[i].reshape(1, H))
    acc = _make_edge_pass(False)(a, bm, sdst2, ssrc2, qcnt,
                                 scale[i], shift[i])[0]

  o = _tc3(acc, deg, h, Wo1, bo1.reshape(1, 32), Wo2, bo2.reshape(1, 16),
           Wo3, bo3.reshape(1, 8))
  return o[:N], batch


# speedup vs baseline: 22.3520x; 1.1558x over previous
import functools

import jax
import jax.numpy as jnp
from jax import lax
from jax.experimental import pallas as pl
from jax.experimental.pallas import tpu as pltpu
from jax.experimental.pallas import tpu_sc as plsc

N = 50000
E = 800000
H = 64

HALF = 25088
NPAD = 2 * HALF
RPT = HALF // 16
TRASH = HALF
ACCR = HALF + 8
C = 64
ET = E // 32
PNCH = (ET + C - 1) // C
ETP = 25104
RW = 32 * ETP
FILLDST = 1 << 30

f32 = jnp.float32
i32 = jnp.int32

_SC_PARAMS = pltpu.CompilerParams(
    use_tc_tiling_on_sc=False, needs_layout_passes=False)


def _mesh():
  return plsc.VectorSubcoreMesh(
      core_axis_name="c", subcore_axis_name="s", num_cores=2, num_subcores=16)


def _lane0(qv, iot16):
  return jnp.sum(jnp.where(iot16 == 0, qv[pl.ds(0, 16)], 0))


@functools.lru_cache(maxsize=None)
def _make_partition():
  out_type = [
      jax.ShapeDtypeStruct((2, RW), i32),
      jax.ShapeDtypeStruct((2, RW), i32),
      jax.ShapeDtypeStruct((64, 16), i32),
  ]
  scratch = [
      pltpu.VMEM((C,), i32),
      pltpu.VMEM((C,), i32),
      pltpu.VMEM((ETP,), i32),
      pltpu.VMEM((ETP,), i32),
      pltpu.VMEM((ETP,), i32),
      pltpu.VMEM((ETP,), i32),
      pltpu.VMEM((16,), i32),
      pltpu.SemaphoreType.DMA,
      pltpu.SemaphoreType.DMA,
  ]

  def body(dst_h, src_h, sdst2, ssrc2, qcnt, dstv, srcv,
           ob0d, ob0s, ob1d, ob1s, qbuf, sem_a, sem_b):
    c = lax.axis_index("c")
    t = lax.axis_index("s")
    w = c * 16 + t
    base = w * ET
    iot = lax.iota(i32, 16)

    def chunk(k, carry):
      n0, n1 = carry
      p0 = pl.multiple_of(base + k * C, 8)
      cp1 = pltpu.async_copy(dst_h.at[pl.ds(p0, C)], dstv, sem_a)
      cp2 = pltpu.async_copy(src_h.at[pl.ds(p0, C)], srcv, sem_b)
      cp1.wait()
      cp2.wait()
      for j in range(C // 16):
        d = dstv[pl.ds(j * 16, 16)]
        s = srcv[pl.ds(j * 16, 16)]
        lpos = (k * C + j * 16) + iot
        vv = lpos < ET
        m0 = vv & (d < HALF)
        m1 = vv & (d >= HALF)
        c0 = plsc.cumsum(m0.astype(i32))
        c1 = plsc.cumsum(m1.astype(i32))
        i0 = (n0 - 1) + c0
        i1 = (n1 - 1) + c1
        plsc.store_scatter(ob0d, [i0], d, mask=m0)
        plsc.store_scatter(ob0s, [i0], s, mask=m0)
        plsc.store_scatter(ob1d, [i1], d, mask=m1)
        plsc.store_scatter(ob1s, [i1], s, mask=m1)
        n0 = n0 + jnp.sum(m0.astype(i32))
        n1 = n1 + jnp.sum(m1.astype(i32))
      return (n0, n1)

    n0, n1 = lax.fori_loop(0, PNCH, chunk, (jnp.array(0, i32),
                                            jnp.array(0, i32)))

    rbase = w * ETP
    pltpu.sync_copy(ob0d, sdst2.at[0, pl.ds(rbase, ETP)])
    pltpu.sync_copy(ob0s, ssrc2.at[0, pl.ds(rbase, ETP)])
    pltpu.sync_copy(ob1d, sdst2.at[1, pl.ds(rbase, ETP)])
    pltpu.sync_copy(ob1s, ssrc2.at[1, pl.ds(rbase, ETP)])

    qbuf[pl.ds(0, 16)] = jnp.where(iot == 0, n0, 0)
    pltpu.sync_copy(qbuf, qcnt.at[w])
    qbuf[pl.ds(0, 16)] = jnp.where(iot == 0, n1, 0)
    pltpu.sync_copy(qbuf, qcnt.at[32 + w])

  return pl.kernel(body, out_type=out_type, mesh=_mesh(),
                   scratch_types=scratch, name="edge_partition",
                   compiler_params=_SC_PARAMS)


@functools.lru_cache(maxsize=None)
def _make_edge_pass(with_deg):
  out_type = [jax.ShapeDtypeStruct((NPAD, H), f32)]
  scratch = [
      pltpu.VMEM((16,), i32),
      pltpu.VMEM((H,), f32),
      pltpu.VMEM((H,), f32),
      [pltpu.VMEM((C,), i32) for _ in range(2)],
      [pltpu.VMEM((C,), i32) for _ in range(2)],
      [pltpu.VMEM((C,), i32) for _ in range(2)],
      [pltpu.VMEM((C,), i32) for _ in range(2)],
      [pltpu.VMEM((C, H), f32) for _ in range(2)],
      [pltpu.VMEM((C, H), f32) for _ in range(2)],
      [pltpu.VMEM((C, H), f32) for _ in range(2)],
      [pltpu.VMEM((C,), i32) for _ in range(2)],
      pltpu.VMEM_SHARED((ACCR, H), f32),
      pltpu.SemaphoreType.DMA,
      pltpu.SemaphoreType.DMA,
      pltpu.SemaphoreType.DMA,
      [pltpu.SemaphoreType.DMA for _ in range(2)],
      [pltpu.SemaphoreType.DMA for _ in range(2)],
  ]
  if with_deg:
    out_type.append(jax.ShapeDtypeStruct((NPAD, H), f32))

  def body(a_h, b_h, sdst2, ssrc2, qcnt, sc_h, sh_h, *refs):
    if with_deg:
      (acc_out, deg_out, qv, scv, shv, dstv, srcv, gidxv, sidxv,
       arow, brow, mbuf, scatidx, acc,
       sem_i, sem_j, sem_s, sem_a, sem_b) = refs
    else:
      (acc_out, qv, scv, shv, dstv, srcv, gidxv, sidxv,
       arow, brow, mbuf, scatidx, acc,
       sem_i, sem_j, sem_s, sem_a, sem_b) = refs
      deg_out = None

    c = lax.axis_index("c")
    t = lax.axis_index("s")
    iot = lax.iota(i32, 16)
    chalf = c * HALF

    pltpu.sync_copy(sc_h, scv)
    pltpu.sync_copy(sh_h, shv)
    scale = [scv[pl.ds(f * 16, 16)] for f in range(4)]
    shift = [shv[pl.ds(f * 16, 16)] for f in range(4)]

    rb0 = (2 * t) * ETP
    rb1 = (2 * t + 1) * ETP
    pltpu.sync_copy(qcnt.at[c * 32 + 2 * t], qv)
    cnt0 = _lane0(qv, iot)
    pltpu.sync_copy(qcnt.at[c * 32 + 2 * t + 1], qv)
    cnt1 = _lane0(qv, iot)
    t0 = (cnt0 + C - 1) // C
    tt = t0 + (cnt1 + C - 1) // C

    def off_cnt(k):
      in0 = k < t0
      koff = jnp.where(in0, k, k - t0) * C
      addr = jnp.where(in0, rb0, rb1) + koff
      cc = jnp.where(in0, cnt0, cnt1)
      return koff, addr, cc

    def load_idx(k, b, make_gidx):
      koff, addr, cc = off_cnt(k)
      p0 = pl.multiple_of(addr, 8)
      cp1 = pltpu.async_copy(sdst2.at[c, pl.ds(p0, C)], dstv[b], sem_i)
      cp2 = pltpu.async_copy(ssrc2.at[c, pl.ds(p0, C)], srcv[b], sem_j)
      cp1.wait()
      cp2.wait()
      for j in range(C // 16):
        sl = pl.ds(j * 16, 16)
        d = dstv[b][sl]
        s = srcv[b][sl]
        valid = ((koff + j * 16) + iot) < cc
        sidxv[b][sl] = jnp.where(valid, d - chalf, TRASH)
        if make_gidx:
          gidxv[b][sl] = jnp.where(valid, d, 0)
          srcv[b][sl] = jnp.where(valid, s, 0)

    def issue_gathers(k, b):
      load_idx(k, b, True)
      pltpu.async_copy(a_h.at[gidxv[b]], arow[b], sem_a[b])
      pltpu.async_copy(b_h.at[srcv[b]], brow[b], sem_b[b])

    def fill_arow(b, val):
      def frow(r, carry):
        for f in range(4):
          arow[b][r, pl.ds(f * 16, 16)] = jnp.full((16,), val, f32)
        return carry
      lax.fori_loop(0, C, frow, 0)

    zbase = t * RPT

    def zero_acc():
      nf, rem = RPT // C, RPT % C
      for i in range(nf):
        pltpu.sync_copy(arow[0], acc.at[pl.ds(zbase + i * C, C)])
      if rem:
        pltpu.sync_copy(arow[0].at[pl.ds(0, rem)],
                        acc.at[pl.ds(zbase + nf * C, rem)])

    if with_deg:
      fill_arow(0, 0.0)
      zero_acc()
      plsc.subcore_barrier()
      fill_arow(0, 1.0)

      def dchunk(k, carry):
        load_idx(k, 0, False)
        pltpu.sync_copy(arow[0], acc.at[sidxv[0]], add=True)
        return carry
      lax.fori_loop(0, tt, dchunk, 0)

      plsc.subcore_barrier()
      pltpu.sync_copy(acc.at[pl.ds(t * RPT, RPT)],
                      deg_out.at[pl.ds(chalf + t * RPT, RPT)])

    fill_arow(0, 0.0)
    zero_acc()
    plsc.subcore_barrier()

    @pl.when(tt > 0)
    def _():
      issue_gathers(0, 0)

    @pl.when(tt > 1)
    def _():
      issue_gathers(1, 1)

    def outer(g, carry):
      for b in range(2):
        k = 2 * g + b

        @pl.when(k < tt)
        def _(b=b, k=k):
          pltpu.make_async_copy(a_h.at[gidxv[b]], arow[b], sem_a[b]).wait()
          pltpu.make_async_copy(b_h.at[srcv[b]], brow[b], sem_b[b]).wait()

          def mrow(r, cc):
            for rr in range(2):
              for f in range(4):
                sl = pl.ds(f * 16, 16)
                y = arow[b][2 * r + rr, sl] + brow[b][2 * r + rr, sl]
                m = jnp.where(y > 0.0, y, jnp.exp(y) - 1.0)
                mbuf[b][2 * r + rr, sl] = m * scale[f] + shift[f]
            return cc
          lax.fori_loop(0, C // 2, mrow, 0)

          @pl.when(k > 0)
          def _():
            pltpu.make_async_copy(
                mbuf[1 - b], acc.at[scatidx[1 - b]], sem_s).wait()

          for j in range(C // 16):
            sl = pl.ds(j * 16, 16)
            scatidx[b][sl] = sidxv[b][sl]
          pltpu.async_copy(mbuf[b], acc.at[scatidx[b]], sem_s, add=True)

          @pl.when(k + 2 < tt)
          def _():
            issue_gathers(k + 2, b)
      return carry
    lax.fori_loop(0, (tt + 1) // 2, outer, 0)

    @pl.when((tt > 0) & (lax.rem(tt - 1, 2) == 0))
    def _():
      pltpu.make_async_copy(mbuf[0], acc.at[scatidx[0]], sem_s).wait()

    @pl.when((tt > 0) & (lax.rem(tt - 1, 2) == 1))
    def _():
      pltpu.make_async_copy(mbuf[1], acc.at[scatidx[1]], sem_s).wait()

    plsc.subcore_barrier()

    pltpu.sync_copy(acc.at[pl.ds(t * RPT, RPT)],
                    acc_out.at[pl.ds(chalf + t * RPT, RPT)])

  return pl.kernel(body, out_type=out_type, mesh=_mesh(),
                   scratch_types=scratch, name="edge_pass",
                   compiler_params=_SC_PARAMS)


_R = 3136
_GRID = NPAD // _R


def _elu(x):
  return jnp.where(x > 0.0, x, jnp.exp(x) - 1.0)


def _rows_spec(w):
  return pl.BlockSpec((_R, w), lambda i: (i, 0))


def _full_spec(r, w):
  return pl.BlockSpec((r, w), lambda i: (0, 0))


def _tc0_body(x_ref, w1_ref, b1_ref, w2_ref, b2_ref, wd_ref, wb_ref, bc_ref,
              h_ref, a_ref, b_ref):
  x = x_ref[...]
  h = _elu(jnp.dot(x, w1_ref[...], preferred_element_type=f32) + b1_ref[...])
  h = _elu(jnp.dot(h, w2_ref[...], preferred_element_type=f32) + b2_ref[...])
  h_ref[...] = h
  a_ref[...] = jnp.dot(h, wd_ref[...], preferred_element_type=f32) + bc_ref[...]
  b_ref[...] = jnp.dot(h, wb_ref[...], preferred_element_type=f32)


def _tc0(xpad, w1, b1, w2, b2, wd, wb, bc_):
  return pl.pallas_call(
      _tc0_body,
      grid=(_GRID,),
      in_specs=[_rows_spec(15), _full_spec(15, H), _full_spec(1, H),
                _full_spec(H, H), _full_spec(1, H), _full_spec(H, H),
                _full_spec(H, H), _full_spec(1, H)],
      out_specs=[_rows_spec(H), _rows_spec(H), _rows_spec(H)],
      out_shape=[jax.ShapeDtypeStruct((NPAD, H), f32)] * 3,
  )(xpad, w1, b1, w2, b2, wd, wb, bc_)


def _tcmid_body(acc_ref, deg_ref, h_ref, wd_ref, wb_ref, bc_ref,
                hn_ref, a_ref, b_ref):
  dinv = 1.0 / jnp.maximum(deg_ref[...][:, 0:1], 1.0)
  hn = acc_ref[...] * dinv + h_ref[...]
  hn_ref[...] = hn
  a_ref[...] = jnp.dot(hn, wd_ref[...], preferred_element_type=f32) + bc_ref[...]
  b_ref[...] = jnp.dot(hn, wb_ref[...], preferred_element_type=f32)


def _tcmid(acc, deg, h, wd, wb, bc_):
  return pl.pallas_call(
      _tcmid_body,
      grid=(_GRID,),
      in_specs=[_rows_spec(H), _rows_spec(H), _rows_spec(H),
                _full_spec(H, H), _full_spec(H, H), _full_spec(1, H)],
      out_specs=[_rows_spec(H), _rows_spec(H), _rows_spec(H)],
      out_shape=[jax.ShapeDtypeStruct((NPAD, H), f32)] * 3,
  )(acc, deg, h, wd, wb, bc_)


def _tc3_body(acc_ref, deg_ref, h_ref, wo1_ref, bo1_ref, wo2_ref, bo2_ref,
              wo3_ref, bo3_ref, o_ref):
  dinv = 1.0 / jnp.maximum(deg_ref[...][:, 0:1], 1.0)
  hn = acc_ref[...] * dinv + h_ref[...]
  o = _elu(jnp.dot(hn, wo1_ref[...], preferred_element_type=f32) + bo1_ref[...])
  o = _elu(jnp.dot(o, wo2_ref[...], preferred_element_type=f32) + bo2_ref[...])
  o_ref[...] = jnp.dot(o, wo3_ref[...], preferred_element_type=f32) + bo3_ref[...]


def _tc3(acc, deg, h, wo1, bo1, wo2, bo2, wo3, bo3):
  return pl.pallas_call(
      _tc3_body,
      grid=(_GRID,),
      in_specs=[_rows_spec(H), _rows_spec(H), _rows_spec(H),
                _full_spec(H, 32), _full_spec(1, 32), _full_spec(32, 16),
                _full_spec(1, 16), _full_spec(16, 8), _full_spec(1, 8)],
      out_specs=[_rows_spec(8)],
      out_shape=[jax.ShapeDtypeStruct((NPAD, 8), f32)],
  )(acc, deg, h, wo1, bo1, wo2, bo2, wo3, bo3)[0]


def kernel(x, edge_index, batch, W_lc1, b_lc1, W_lc2, b_lc2, Wc, bc, gc,
           betac, rmc, rvc, Wo1, bo1, Wo2, bo2, Wo3, bo3):
  dstp = jnp.pad(edge_index[1], (0, 128), constant_values=FILLDST)
  srcp = jnp.pad(edge_index[0], (0, 128))
  sdst2, ssrc2, qcnt = _make_partition()(dstp, srcp)

  xpad = jnp.pad(x, ((0, NPAD - N), (0, 0)))
  scale = gc / jnp.sqrt(rvc + 1e-5)
  shift = betac - rmc * scale
  wd = Wc[:, :H, :] - Wc[:, H:, :]
  wb = Wc[:, H:, :]

  h, a, bm = _tc0(xpad, W_lc1, b_lc1.reshape(1, H), W_lc2,
                  b_lc2.reshape(1, H), wd[0], wb[0], bc[0].reshape(1, H))
  acc, deg = _make_edge_pass(True)(a, bm, sdst2, ssrc2, qcnt,
                                   scale[0], shift[0])
  for i in (1, 2):
    h, a, bm = _tcmid(acc, deg, h, wd[i], wb[i], bc[i].reshape(1, H))
    acc = _make_edge_pass(False)(a, bm, sdst2, ssrc2, qcnt,
                                 scale[i], shift[i])[0]

  o = _tc3(acc, deg, h, Wo1, bo1.reshape(1, 32), Wo2, bo2.reshape(1, 16),
           Wo3, bo3.reshape(1, 8))
  return o[:N], batch


# speedup vs baseline: 23.6217x; 1.0568x over previous
import functools

import jax
import jax.numpy as jnp
from jax import lax
from jax.experimental import pallas as pl
from jax.experimental.pallas import tpu as pltpu
from jax.experimental.pallas import tpu_sc as plsc

N = 50000
E = 800000
H = 64

HALF = 25088
NPAD = 2 * HALF
RPT = HALF // 16
TRASH = HALF
ACCR = HALF + 8
C = 64
ET = E // 32
PNCH = (ET + C - 1) // C
ETP = 25104
RW = 32 * ETP
FILLDST = 1 << 30

f32 = jnp.float32
i32 = jnp.int32

_SC_PARAMS = pltpu.CompilerParams(
    use_tc_tiling_on_sc=False, needs_layout_passes=False)


def _mesh():
  return plsc.VectorSubcoreMesh(
      core_axis_name="c", subcore_axis_name="s", num_cores=2, num_subcores=16)


def _lane0(qv, iot16):
  return jnp.sum(jnp.where(iot16 == 0, qv[pl.ds(0, 16)], 0))


@functools.lru_cache(maxsize=None)
def _make_partition():
  out_type = [
      jax.ShapeDtypeStruct((2, RW), i32),
      jax.ShapeDtypeStruct((2, RW), i32),
      jax.ShapeDtypeStruct((64, 16), i32),
  ]
  scratch = [
      [pltpu.VMEM((C,), i32) for _ in range(2)],
      [pltpu.VMEM((C,), i32) for _ in range(2)],
      pltpu.VMEM((ETP,), i32),
      pltpu.VMEM((ETP,), i32),
      pltpu.VMEM((ETP,), i32),
      pltpu.VMEM((ETP,), i32),
      pltpu.VMEM((16,), i32),
      pltpu.SemaphoreType.DMA,
      pltpu.SemaphoreType.DMA,
  ]

  def body(dst_h, src_h, sdst2, ssrc2, qcnt, dstv, srcv,
           ob0d, ob0s, ob1d, ob1s, qbuf, sem_a, sem_b):
    c = lax.axis_index("c")
    t = lax.axis_index("s")
    w = c * 16 + t
    base = w * ET
    iot = lax.iota(i32, 16)

    def idx_start(k, b):
      p0 = pl.multiple_of(base + k * C, 8)
      pltpu.async_copy(dst_h.at[pl.ds(p0, C)], dstv[b], sem_a)
      pltpu.async_copy(src_h.at[pl.ds(p0, C)], srcv[b], sem_b)

    def idx_wait(k, b):
      p0 = pl.multiple_of(base + k * C, 8)
      pltpu.make_async_copy(dst_h.at[pl.ds(p0, C)], dstv[b], sem_a).wait()
      pltpu.make_async_copy(src_h.at[pl.ds(p0, C)], srcv[b], sem_b).wait()

    def process(k, b, n0, n1, issue_next):
      idx_wait(k, b)
      if issue_next:
        idx_start(jnp.minimum(k + 1, PNCH - 1), 1 - b)
      for j in range(C // 16):
        d = dstv[b][pl.ds(j * 16, 16)]
        s = srcv[b][pl.ds(j * 16, 16)]
        lpos = (k * C + j * 16) + iot
        vv = lpos < ET
        m0 = vv & (d < HALF)
        m1 = vv & (d >= HALF)
        c0 = plsc.cumsum(m0.astype(i32))
        c1 = plsc.cumsum(m1.astype(i32))
        i0 = (n0 - 1) + c0
        i1 = (n1 - 1) + c1
        plsc.store_scatter(ob0d, [i0], d, mask=m0)
        plsc.store_scatter(ob0s, [i0], s, mask=m0)
        plsc.store_scatter(ob1d, [i1], d, mask=m1)
        plsc.store_scatter(ob1s, [i1], s, mask=m1)
        n0 = n0 + jnp.sum(m0.astype(i32))
        n1 = n1 + jnp.sum(m1.astype(i32))
      return n0, n1

    idx_start(0, 0)

    def outer(g, carry):
      n0, n1 = carry
      for b in range(2):
        n0, n1 = process(2 * g + b, b, n0, n1, True)
      return (n0, n1)

    n0, n1 = lax.fori_loop(0, PNCH // 2, outer, (jnp.array(0, i32),
                                                 jnp.array(0, i32)))
    if PNCH % 2:
      n0, n1 = process(PNCH - 1, 0, n0, n1, False)
    else:
      idx_wait(PNCH - 1, 0)

    rbase = w * ETP
    pltpu.sync_copy(ob0d, sdst2.at[0, pl.ds(rbase, ETP)])
    pltpu.sync_copy(ob0s, ssrc2.at[0, pl.ds(rbase, ETP)])
    pltpu.sync_copy(ob1d, sdst2.at[1, pl.ds(rbase, ETP)])
    pltpu.sync_copy(ob1s, ssrc2.at[1, pl.ds(rbase, ETP)])

    qbuf[pl.ds(0, 16)] = jnp.where(iot == 0, n0, 0)
    pltpu.sync_copy(qbuf, qcnt.at[w])
    qbuf[pl.ds(0, 16)] = jnp.where(iot == 0, n1, 0)
    pltpu.sync_copy(qbuf, qcnt.at[32 + w])

  return pl.kernel(body, out_type=out_type, mesh=_mesh(),
                   scratch_types=scratch, name="edge_partition",
                   compiler_params=_SC_PARAMS)


@functools.lru_cache(maxsize=None)
def _make_edge_pass(with_deg):
  out_type = [jax.ShapeDtypeStruct((NPAD, H), f32)]
  scratch = [
      pltpu.VMEM((16,), i32),
      pltpu.VMEM((H,), f32),
      pltpu.VMEM((H,), f32),
      [pltpu.VMEM((C,), i32) for _ in range(2)],
      [pltpu.VMEM((C,), i32) for _ in range(2)],
      [pltpu.VMEM((C,), i32) for _ in range(2)],
      [pltpu.VMEM((C,), i32) for _ in range(2)],
      [pltpu.VMEM((C, H), f32) for _ in range(2)],
      [pltpu.VMEM((C, H), f32) for _ in range(2)],
      [pltpu.VMEM((C, H), f32) for _ in range(2)],
      [pltpu.VMEM((C,), i32) for _ in range(2)],
      pltpu.VMEM_SHARED((ACCR, H), f32),
      pltpu.SemaphoreType.DMA,
      pltpu.SemaphoreType.DMA,
      pltpu.SemaphoreType.DMA,
      [pltpu.SemaphoreType.DMA for _ in range(2)],
      [pltpu.SemaphoreType.DMA for _ in range(2)],
  ]
  if with_deg:
    out_type.append(jax.ShapeDtypeStruct((NPAD, H), f32))

  def body(a_h, b_h, sdst2, ssrc2, qcnt, sc_h, sh_h, *refs):
    if with_deg:
      (acc_out, deg_out, qv, scv, shv, dstv, srcv, gidxv, sidxv,
       arow, brow, mbuf, scatidx, acc,
       sem_i, sem_j, sem_s, sem_a, sem_b) = refs
    else:
      (acc_out, qv, scv, shv, dstv, srcv, gidxv, sidxv,
       arow, brow, mbuf, scatidx, acc,
       sem_i, sem_j, sem_s, sem_a, sem_b) = refs
      deg_out = None

    c = lax.axis_index("c")
    t = lax.axis_index("s")
    iot = lax.iota(i32, 16)
    chalf = c * HALF

    pltpu.sync_copy(sc_h, scv)
    pltpu.sync_copy(sh_h, shv)
    scale = [scv[pl.ds(f * 16, 16)] for f in range(4)]
    shift = [shv[pl.ds(f * 16, 16)] for f in range(4)]

    rb0 = (2 * t) * ETP
    rb1 = (2 * t + 1) * ETP
    pltpu.sync_copy(qcnt.at[c * 32 + 2 * t], qv)
    cnt0 = _lane0(qv, iot)
    pltpu.sync_copy(qcnt.at[c * 32 + 2 * t + 1], qv)
    cnt1 = _lane0(qv, iot)
    t0 = (cnt0 + C - 1) // C
    tt = t0 + (cnt1 + C - 1) // C

    def off_cnt(k):
      in0 = k < t0
      koff = jnp.where(in0, k, k - t0) * C
      addr = jnp.where(in0, rb0, rb1) + koff
      cc = jnp.where(in0, cnt0, cnt1)
      return koff, addr, cc

    def load_idx(k, b, make_gidx):
      koff, addr, cc = off_cnt(k)
      p0 = pl.multiple_of(addr, 8)
      cp1 = pltpu.async_copy(sdst2.at[c, pl.ds(p0, C)], dstv[b], sem_i)
      cp2 = pltpu.async_copy(ssrc2.at[c, pl.ds(p0, C)], srcv[b], sem_j)
      cp1.wait()
      cp2.wait()
      for j in range(C // 16):
        sl = pl.ds(j * 16, 16)
        d = dstv[b][sl]
        s = srcv[b][sl]
        valid = ((koff + j * 16) + iot) < cc
        sidxv[b][sl] = jnp.where(valid, d - chalf, TRASH)
        if make_gidx:
          gidxv[b][sl] = jnp.where(valid, d, 0)
          srcv[b][sl] = jnp.where(valid, s, 0)

    def issue_gathers(k, b):
      load_idx(k, b, True)
      pltpu.async_copy(a_h.at[gidxv[b]], arow[b], sem_a[b])
      pltpu.async_copy(b_h.at[srcv[b]], brow[b], sem_b[b])

    def fill_arow(b, val):
      def frow(r, carry):
        for f in range(4):
          arow[b][r, pl.ds(f * 16, 16)] = jnp.full((16,), val, f32)
        return carry
      lax.fori_loop(0, C, frow, 0)

    zbase = t * RPT

    def zero_acc():
      nf, rem = RPT // C, RPT % C
      for i in range(nf):
        pltpu.sync_copy(arow[0], acc.at[pl.ds(zbase + i * C, C)])
      if rem:
        pltpu.sync_copy(arow[0].at[pl.ds(0, rem)],
                        acc.at[pl.ds(zbase + nf * C, rem)])

    if with_deg:
      fill_arow(0, 0.0)
      zero_acc()
      plsc.subcore_barrier()
      fill_arow(0, 1.0)
      fill_arow(1, 1.0)

      def didx_start(k, b):
        _, addr, _ = off_cnt(k)
        pltpu.async_copy(
            sdst2.at[c, pl.ds(pl.multiple_of(addr, 8), C)], dstv[b], sem_i)

      @pl.when(tt > 0)
      def _():
        didx_start(0, 0)

      def douter(g, carry):
        for b in range(2):
          k = 2 * g + b

          @pl.when(k < tt)
          def _(b=b, k=k):
            koff, addr, cc = off_cnt(k)
            pltpu.make_async_copy(
                sdst2.at[c, pl.ds(pl.multiple_of(addr, 8), C)],
                dstv[b], sem_i).wait()

            @pl.when(k + 1 < tt)
            def _():
              didx_start(k + 1, 1 - b)

            for j in range(C // 16):
              sl = pl.ds(j * 16, 16)
              valid = ((koff + j * 16) + iot) < cc
              sidxv[b][sl] = jnp.where(valid, dstv[b][sl] - chalf, TRASH)
            pltpu.sync_copy(arow[b], acc.at[sidxv[b]], add=True)
        return carry
      lax.fori_loop(0, (tt + 1) // 2, douter, 0)

      plsc.subcore_barrier()
      pltpu.sync_copy(acc.at[pl.ds(t * RPT, RPT)],
                      deg_out.at[pl.ds(chalf + t * RPT, RPT)])

    fill_arow(0, 0.0)
    zero_acc()
    plsc.subcore_barrier()

    @pl.when(tt > 0)
    def _():
      issue_gathers(0, 0)

    @pl.when(tt > 1)
    def _():
      issue_gathers(1, 1)

    def outer(g, carry):
      for b in range(2):
        k = 2 * g + b

        @pl.when(k < tt)
        def _(b=b, k=k):
          pltpu.make_async_copy(a_h.at[gidxv[b]], arow[b], sem_a[b]).wait()
          pltpu.make_async_copy(b_h.at[srcv[b]], brow[b], sem_b[b]).wait()

          def mrow(r, cc):
            for rr in range(2):
              for f in range(4):
                sl = pl.ds(f * 16, 16)
                y = arow[b][2 * r + rr, sl] + brow[b][2 * r + rr, sl]
                m = jnp.where(y > 0.0, y, jnp.exp(y) - 1.0)
                mbuf[b][2 * r + rr, sl] = m * scale[f] + shift[f]
            return cc
          lax.fori_loop(0, C // 2, mrow, 0)

          @pl.when(k > 0)
          def _():
            pltpu.make_async_copy(
                mbuf[1 - b], acc.at[scatidx[1 - b]], sem_s).wait()

          for j in range(C // 16):
            sl = pl.ds(j * 16, 16)
            scatidx[b][sl] = sidxv[b][sl]
          pltpu.async_copy(mbuf[b], acc.at[scatidx[b]], sem_s, add=True)

          @pl.when(k + 2 < tt)
          def _():
            issue_gathers(k + 2, b)
      return carry
    lax.fori_loop(0, (tt + 1) // 2, outer, 0)

    @pl.when((tt > 0) & (lax.rem(tt - 1, 2) == 0))
    def _():
      pltpu.make_async_copy(mbuf[0], acc.at[scatidx[0]], sem_s).wait()

    @pl.when((tt > 0) & (lax.rem(tt - 1, 2) == 1))
    def _():
      pltpu.make_async_copy(mbuf[1], acc.at[scatidx[1]], sem_s).wait()

    plsc.subcore_barrier()

    pltpu.sync_copy(acc.at[pl.ds(t * RPT, RPT)],
                    acc_out.at[pl.ds(chalf + t * RPT, RPT)])

  return pl.kernel(body, out_type=out_type, mesh=_mesh(),
                   scratch_types=scratch, name="edge_pass",
                   compiler_params=_SC_PARAMS)


_R = 3136
_GRID = NPAD // _R


def _elu(x):
  return jnp.where(x > 0.0, x, jnp.exp(x) - 1.0)


def _rows_spec(w):
  return pl.BlockSpec((_R, w), lambda i: (i, 0))


def _full_spec(r, w):
  return pl.BlockSpec((r, w), lambda i: (0, 0))


def _tc0_body(x_ref, w1_ref, b1_ref, w2_ref, b2_ref, wd_ref, wb_ref, bc_ref,
              h_ref, a_ref, b_ref):
  x = x_ref[...]
  h = _elu(jnp.dot(x, w1_ref[...], preferred_element_type=f32) + b1_ref[...])
  h = _elu(jnp.dot(h, w2_ref[...], preferred_element_type=f32) + b2_ref[...])
  h_ref[...] = h
  a_ref[...] = jnp.dot(h, wd_ref[...], preferred_element_type=f32) + bc_ref[...]
  b_ref[...] = jnp.dot(h, wb_ref[...], preferred_element_type=f32)


def _tc0(xpad, w1, b1, w2, b2, wd, wb, bc_):
  return pl.pallas_call(
      _tc0_body,
      grid=(_GRID,),
      in_specs=[_rows_spec(15), _full_spec(15, H), _full_spec(1, H),
                _full_spec(H, H), _full_spec(1, H), _full_spec(H, H),
                _full_spec(H, H), _full_spec(1, H)],
      out_specs=[_rows_spec(H), _rows_spec(H), _rows_spec(H)],
      out_shape=[jax.ShapeDtypeStruct((NPAD, H), f32)] * 3,
  )(xpad, w1, b1, w2, b2, wd, wb, bc_)


def _tcmid_body(acc_ref, deg_ref, h_ref, wd_ref, wb_ref, bc_ref,
                hn_ref, a_ref, b_ref):
  dinv = 1.0 / jnp.maximum(deg_ref[...][:, 0:1], 1.0)
  hn = acc_ref[...] * dinv + h_ref[...]
  hn_ref[...] = hn
  a_ref[...] = jnp.dot(hn, wd_ref[...], preferred_element_type=f32) + bc_ref[...]
  b_ref[...] = jnp.dot(hn, wb_ref[...], preferred_element_type=f32)


def _tcmid(acc, deg, h, wd, wb, bc_):
  return pl.pallas_call(
      _tcmid_body,
      grid=(_GRID,),
      in_specs=[_rows_spec(H), _rows_spec(H), _rows_spec(H),
                _full_spec(H, H), _full_spec(H, H), _full_spec(1, H)],
      out_specs=[_rows_spec(H), _rows_spec(H), _rows_spec(H)],
      out_shape=[jax.ShapeDtypeStruct((NPAD, H), f32)] * 3,
  )(acc, deg, h, wd, wb, bc_)


def _tc3_body(acc_ref, deg_ref, h_ref, wo1_ref, bo1_ref, wo2_ref, bo2_ref,
              wo3_ref, bo3_ref, o_ref):
  dinv = 1.0 / jnp.maximum(deg_ref[...][:, 0:1], 1.0)
  hn = acc_ref[...] * dinv + h_ref[...]
  o = _elu(jnp.dot(hn, wo1_ref[...], preferred_element_type=f32) + bo1_ref[...])
  o = _elu(jnp.dot(o, wo2_ref[...], preferred_element_type=f32) + bo2_ref[...])
  o_ref[...] = jnp.dot(o, wo3_ref[...], preferred_element_type=f32) + bo3_ref[...]


def _tc3(acc, deg, h, wo1, bo1, wo2, bo2, wo3, bo3):
  return pl.pallas_call(
      _tc3_body,
      grid=(_GRID,),
      in_specs=[_rows_spec(H), _rows_spec(H), _rows_spec(H),
                _full_spec(H, 32), _full_spec(1, 32), _full_spec(32, 16),
                _full_spec(1, 16), _full_spec(16, 8), _full_spec(1, 8)],
      out_specs=[_rows_spec(8)],
      out_shape=[jax.ShapeDtypeStruct((NPAD, 8), f32)],
  )(acc, deg, h, wo1, bo1, wo2, bo2, wo3, bo3)[0]


def kernel(x, edge_index, batch, W_lc1, b_lc1, W_lc2, b_lc2, Wc, bc, gc,
           betac, rmc, rvc, Wo1, bo1, Wo2, bo2, Wo3, bo3):
  dstp = jnp.pad(edge_index[1], (0, 128), constant_values=FILLDST)
  srcp = jnp.pad(edge_index[0], (0, 128))
  sdst2, ssrc2, qcnt = _make_partition()(dstp, srcp)

  xpad = jnp.pad(x, ((0, NPAD - N), (0, 0)))
  scale = gc / jnp.sqrt(rvc + 1e-5)
  shift = betac - rmc * scale
  wd = Wc[:, :H, :] - Wc[:, H:, :]
  wb = Wc[:, H:, :]

  h, a, bm = _tc0(xpad, W_lc1, b_lc1.reshape(1, H), W_lc2,
                  b_lc2.reshape(1, H), wd[0], wb[0], bc[0].reshape(1, H))
  acc, deg = _make_edge_pass(True)(a, bm, sdst2, ssrc2, qcnt,
                                   scale[0], shift[0])
  for i in (1, 2):
    h, a, bm = _tcmid(acc, deg, h, wd[i], wb[i], bc[i].reshape(1, H))
    acc = _make_edge_pass(False)(a, bm, sdst2, ssrc2, qcnt,
                                 scale[i], shift[i])[0]

  o = _tc3(acc, deg, h, Wo1, bo1.reshape(1, 32), Wo2, bo2.reshape(1, 16),
           Wo3, bo3.reshape(1, 8))
  return o[:N], batch
